# baseline (speedup 1.0000x reference)
import functools

import jax
import jax.numpy as jnp
from jax import lax
from jax.experimental import pallas as pl
from jax.experimental.pallas import tpu as pltpu
from jax.experimental.pallas import tpu_sc as plsc

NC = 2
NS = 16
NW = NC * NS
L = 16
CR = 128
W = 128


@functools.lru_cache(maxsize=None)
def _make_transpose_kernel(nv, d):
    n_full = nv // W
    tail = nv - n_full * W
    n_iter = (n_full + NW - 1) // NW
    mesh = plsc.VectorSubcoreMesh(core_axis_name="c", subcore_axis_name="s")
    cp = pltpu.CompilerParams(needs_layout_passes=False)

    @functools.partial(
        pl.kernel,
        compiler_params=cp,
        out_type=jax.ShapeDtypeStruct((nv // 2, W), jnp.float32),
        mesh=mesh,
        scratch_types=[
            pltpu.VMEM((2, d, W), jnp.float32),
            pltpu.VMEM((2, W // 2, W), jnp.float32),
            pltpu.SemaphoreType.DMA,
            pltpu.SemaphoreType.DMA,
            pltpu.SemaphoreType.DMA,
            pltpu.SemaphoreType.DMA,
        ],
    )
    def k(ent_t_hbm, tail_hbm, out_hbm, buf_v, bt_v, si0, si1, so0, so1):
        wid = lax.axis_index("s") * NC + lax.axis_index("c")
        iota = lax.iota(jnp.int32, L)
        r0 = iota >> 1
        colb = (iota & 1) * d
        si = [si0, si1]
        so = [so0, so1]

        def col_of(i):
            return wid + i * NW

        def fire_in(i, pp):
            @pl.when(col_of(i) < n_full)
            def _():
                pltpu.async_copy(
                    ent_t_hbm.at[:, pl.ds(col_of(i) * W, W)],
                    buf_v.at[pp], si[pp])

        def step(i, pp, wait_out):
            c = col_of(i)

            @pl.when(c < n_full)
            def _():
                pltpu.make_async_copy(
                    ent_t_hbm.at[:, pl.ds(c * W, W)], buf_v.at[pp],
                    si[pp]).wait()
                if wait_out:
                    pltpu.make_async_copy(
                        bt_v.at[pp], out_hbm.at[pl.ds(0, W // 2)],
                        so[pp]).wait()
                for dd in range(d):
                    cols = colb + dd
                    for eg in range(W // L):
                        plsc.store_scatter(
                            bt_v.at[pp], [eg * (L // 2) + r0, cols],
                            buf_v[pp, dd, pl.ds(eg * L, L)])
                pltpu.async_copy(
                    bt_v.at[pp], out_hbm.at[pl.ds(c * (W // 2), W // 2)],
                    so[pp])
                fire_in(i + 2, pp)

        fire_in(0, 0)
        fire_in(1, 1)
        step(0, 0, False)
        step(1, 1, False)

        @pl.loop(1, (n_iter + 1) // 2)
        def _(i2):
            step(2 * i2, 0, True)
            step(2 * i2 + 1, 1, True)

        for pp in range(2):
            pltpu.make_async_copy(
                bt_v.at[pp], out_hbm.at[pl.ds(0, W // 2)], so[pp]).wait()

        if tail:
            @pl.when(wid == 1)
            def _():
                pltpu.async_copy(tail_hbm, buf_v.at[0, pl.ds(0, tail // 2)],
                                 si[0]).wait()
                pltpu.async_copy(buf_v.at[0, pl.ds(0, tail // 2)],
                                 out_hbm.at[pl.ds(n_full * (W // 2),
                                                  tail // 2)], si[0]).wait()

    return k


@functools.lru_cache(maxsize=None)
def _make_sc_kernel(batch, d):
    per_w = batch // NW
    n_chunks = per_w // CR
    mesh = plsc.VectorSubcoreMesh(core_axis_name="c", subcore_axis_name="s")
    cp = pltpu.CompilerParams(needs_layout_passes=False)

    @functools.partial(
        pl.kernel,
        compiler_params=cp,
        out_type=jax.ShapeDtypeStruct((batch,), jnp.float32),
        mesh=mesh,
        scratch_types=[
            pltpu.VMEM((per_w,), jnp.int32),
            pltpu.VMEM((per_w,), jnp.int32),
            pltpu.VMEM((per_w,), jnp.int32),
            pltpu.VMEM((per_w,), jnp.int32),
            pltpu.VMEM((per_w,), jnp.int32),
            pltpu.VMEM((CR, W), jnp.float32),
            pltpu.VMEM((CR, W), jnp.float32),
            pltpu.VMEM((CR, W), jnp.float32),
            pltpu.VMEM((per_w,), jnp.float32),
            pltpu.SemaphoreType.DMA,
        ],
    )
    def k(head_hbm, rel_hbm, tail_hbm, ent_hbm, relemb_hbm, out_hbm,
          hi_v, ri_v, ti_v, hq_v, tq_v, h_v, r_v, t_v, o_v, sem):
        wid = lax.axis_index("s") * NC + lax.axis_index("c")
        base = wid * per_w

        pltpu.sync_copy(head_hbm.at[pl.ds(base, per_w)], hi_v)
        pltpu.sync_copy(rel_hbm.at[pl.ds(base, per_w)], ri_v)
        pltpu.sync_copy(tail_hbm.at[pl.ds(base, per_w)], ti_v)

        @pl.loop(0, per_w // L)
        def _(i):
            sl = pl.ds(i * L, L)
            hq_v[sl] = hi_v[sl] >> 1
            tq_v[sl] = ti_v[sl] >> 1

        iota = lax.iota(jnp.int32, L)

        @pl.loop(0, n_chunks)
        def _(g):
            cps = [
                pltpu.async_copy(ent_hbm.at[hq_v.at[pl.ds(g * CR, CR)]], h_v, sem),
                pltpu.async_copy(relemb_hbm.at[ri_v.at[pl.ds(g * CR, CR)]], r_v, sem),
                pltpu.async_copy(ent_hbm.at[tq_v.at[pl.ds(g * CR, CR)]], t_v, sem),
            ]
            for cpd in cps:
                cpd.wait()

            for eg in range(CR // L):
                sl = pl.ds(g * CR + eg * L, L)
                rows = eg * L + iota
                hc = (hi_v[sl] & 1) * d
                tc = (ti_v[sl] & 1) * d
                acc = jnp.zeros((L,), jnp.float32)
                for dd in range(d):
                    hv = plsc.load_gather(h_v, [rows, hc + dd])
                    rv = plsc.load_gather(r_v, [rows, jnp.full((L,), dd, jnp.int32)])
                    tv = plsc.load_gather(t_v, [rows, tc + dd])
                    acc = acc + hv * rv * tv
                o_v[sl] = acc

        pltpu.sync_copy(o_v, out_hbm.at[pl.ds(base, per_w)])

    return k


def kernel(head, relation, tail, entity_embeddings, relation_embeddings):
    batch = head.shape[0]
    nv, d = entity_embeddings.shape
    ent_t = jnp.swapaxes(entity_embeddings, 0, 1)
    n_full = nv // W
    tail_p = entity_embeddings[n_full * W:].reshape(-1, W)
    ent_pairs = _make_transpose_kernel(nv, d)(ent_t, tail_p)
    rel_p = jnp.pad(relation_embeddings, ((0, 0), (0, W - d)))
    k = _make_sc_kernel(batch, d)
    return k(head.astype(jnp.int32), relation.astype(jnp.int32),
             tail.astype(jnp.int32), ent_pairs, rel_p)


# speedup vs baseline: 1.2626x; 1.2626x over previous
import functools

import jax
import jax.numpy as jnp
from jax import lax
from jax.experimental import pallas as pl
from jax.experimental.pallas import tpu as pltpu
from jax.experimental.pallas import tpu_sc as plsc

NC = 2
NS = 16
NW = NC * NS
L = 16
CR = 128
W = 128


@functools.lru_cache(maxsize=None)
def _make_transpose_kernel(nv, d):
    n_full = nv // W
    tail = nv - n_full * W
    n_iter = (n_full + NW - 1) // NW
    mesh = plsc.VectorSubcoreMesh(core_axis_name="c", subcore_axis_name="s")
    cp = pltpu.CompilerParams(needs_layout_passes=False)

    @functools.partial(
        pl.kernel,
        compiler_params=cp,
        out_type=jax.ShapeDtypeStruct((nv // 2, W), jnp.float32),
        mesh=mesh,
        scratch_types=[
            pltpu.VMEM((2, d, W), jnp.float32),
            pltpu.VMEM((2, W // 2, W), jnp.float32),
            pltpu.SemaphoreType.DMA,
            pltpu.SemaphoreType.DMA,
            pltpu.SemaphoreType.DMA,
            pltpu.SemaphoreType.DMA,
        ],
    )
    def k(ent_t_hbm, tail_hbm, out_hbm, buf_v, bt_v, si0, si1, so0, so1):
        wid = lax.axis_index("s") * NC + lax.axis_index("c")
        iota = lax.iota(jnp.int32, L)
        r0 = iota >> 1
        colb = (iota & 1) * d
        rows8 = [eg * (L // 2) + r0 for eg in range(W // L)]
        cols64 = [colb + dd for dd in range(d)]
        si = [si0, si1]
        so = [so0, so1]

        def col_of(i):
            return wid + i * NW

        def fire_in(i, pp):
            @pl.when(col_of(i) < n_full)
            def _():
                pltpu.async_copy(
                    ent_t_hbm.at[:, pl.ds(col_of(i) * W, W)],
                    buf_v.at[pp], si[pp])

        def step(i, pp, wait_out):
            c = col_of(i)

            @pl.when(c < n_full)
            def _():
                pltpu.make_async_copy(
                    ent_t_hbm.at[:, pl.ds(c * W, W)], buf_v.at[pp],
                    si[pp]).wait()
                if wait_out:
                    pltpu.make_async_copy(
                        bt_v.at[pp], out_hbm.at[pl.ds(0, W // 2)],
                        so[pp]).wait()
                for eg in range(W // L):
                    rows = rows8[eg]
                    for d0 in range(0, d, 8):
                        vals = [buf_v[pp, d0 + j, pl.ds(eg * L, L)]
                                for j in range(8)]
                        for j in range(8):
                            plsc.store_scatter(
                                bt_v.at[pp], [rows, cols64[d0 + j]], vals[j])
                pltpu.async_copy(
                    bt_v.at[pp], out_hbm.at[pl.ds(c * (W // 2), W // 2)],
                    so[pp])
                fire_in(i + 2, pp)

        fire_in(0, 0)
        fire_in(1, 1)
        step(0, 0, False)
        step(1, 1, False)

        @pl.loop(1, (n_iter + 1) // 2)
        def _(i2):
            step(2 * i2, 0, True)
            step(2 * i2 + 1, 1, True)

        for pp in range(2):
            pltpu.make_async_copy(
                bt_v.at[pp], out_hbm.at[pl.ds(0, W // 2)], so[pp]).wait()

        if tail:
            @pl.when(wid == 1)
            def _():
                pltpu.async_copy(tail_hbm, buf_v.at[0, pl.ds(0, tail // 2)],
                                 si[0]).wait()
                pltpu.async_copy(buf_v.at[0, pl.ds(0, tail // 2)],
                                 out_hbm.at[pl.ds(n_full * (W // 2),
                                                  tail // 2)], si[0]).wait()

    return k


@functools.lru_cache(maxsize=None)
def _make_sc_kernel(batch, d):
    per_w = batch // NW
    n_chunks = per_w // CR
    mesh = plsc.VectorSubcoreMesh(core_axis_name="c", subcore_axis_name="s")
    cp = pltpu.CompilerParams(needs_layout_passes=False)

    @functools.partial(
        pl.kernel,
        compiler_params=cp,
        out_type=jax.ShapeDtypeStruct((batch,), jnp.float32),
        mesh=mesh,
        scratch_types=[
            pltpu.VMEM((per_w,), jnp.int32),
            pltpu.VMEM((per_w,), jnp.int32),
            pltpu.VMEM((per_w,), jnp.int32),
            pltpu.VMEM((per_w,), jnp.int32),
            pltpu.VMEM((per_w,), jnp.int32),
            pltpu.VMEM((CR, W), jnp.float32),
            pltpu.VMEM((CR, W), jnp.float32),
            pltpu.VMEM((CR, W), jnp.float32),
            pltpu.VMEM((per_w,), jnp.float32),
            pltpu.SemaphoreType.DMA,
        ],
    )
    def k(head_hbm, rel_hbm, tail_hbm, ent_hbm, relemb_hbm, out_hbm,
          hi_v, ri_v, ti_v, hq_v, tq_v, h_v, r_v, t_v, o_v, sem):
        wid = lax.axis_index("s") * NC + lax.axis_index("c")
        base = wid * per_w

        pltpu.sync_copy(head_hbm.at[pl.ds(base, per_w)], hi_v)
        pltpu.sync_copy(rel_hbm.at[pl.ds(base, per_w)], ri_v)
        pltpu.sync_copy(tail_hbm.at[pl.ds(base, per_w)], ti_v)

        @pl.loop(0, per_w // L)
        def _(i):
            sl = pl.ds(i * L, L)
            hq_v[sl] = hi_v[sl] >> 1
            tq_v[sl] = ti_v[sl] >> 1

        iota = lax.iota(jnp.int32, L)

        @pl.loop(0, n_chunks)
        def _(g):
            cps = [
                pltpu.async_copy(ent_hbm.at[hq_v.at[pl.ds(g * CR, CR)]], h_v, sem),
                pltpu.async_copy(relemb_hbm.at[ri_v.at[pl.ds(g * CR, CR)]], r_v, sem),
                pltpu.async_copy(ent_hbm.at[tq_v.at[pl.ds(g * CR, CR)]], t_v, sem),
            ]
            for cpd in cps:
                cpd.wait()

            for eg in range(CR // L):
                sl = pl.ds(g * CR + eg * L, L)
                rows = eg * L + iota
                hc = (hi_v[sl] & 1) * d
                tc = (ti_v[sl] & 1) * d
                accs = [jnp.zeros((L,), jnp.float32) for _ in range(4)]
                for dd in range(d):
                    hv = plsc.load_gather(h_v, [rows, hc + dd])
                    rv = plsc.load_gather(r_v, [rows, jnp.full((L,), dd, jnp.int32)])
                    tv = plsc.load_gather(t_v, [rows, tc + dd])
                    accs[dd % 4] = accs[dd % 4] + hv * rv * tv
                o_v[sl] = (accs[0] + accs[1]) + (accs[2] + accs[3])

        pltpu.sync_copy(o_v, out_hbm.at[pl.ds(base, per_w)])

    return k


def kernel(head, relation, tail, entity_embeddings, relation_embeddings):
    batch = head.shape[0]
    nv, d = entity_embeddings.shape
    ent_t = jnp.swapaxes(entity_embeddings, 0, 1)
    n_full = nv // W
    tail_p = entity_embeddings[n_full * W:].reshape(-1, W)
    ent_pairs = _make_transpose_kernel(nv, d)(ent_t, tail_p)
    rel_p = jnp.pad(relation_embeddings, ((0, 0), (0, W - d)))
    k = _make_sc_kernel(batch, d)
    return k(head.astype(jnp.int32), relation.astype(jnp.int32),
             tail.astype(jnp.int32), ent_pairs, rel_p)


# speedup vs baseline: 2.0632x; 1.6341x over previous
import functools

import jax
import jax.numpy as jnp
from jax import lax
from jax.experimental import pallas as pl
from jax.experimental.pallas import tpu as pltpu
from jax.experimental.pallas import tpu_sc as plsc

NC = 2
NS = 16
NW = NC * NS
L = 16
IDX_CHUNK = 128


@functools.lru_cache(maxsize=None)
def _make_sc_kernel(batch, d):
    per_w = batch // NW
    chunks = per_w // IDX_CHUNK
    mesh = plsc.VectorSubcoreMesh(core_axis_name="c", subcore_axis_name="s")
    cp = pltpu.CompilerParams(needs_layout_passes=False,
                              use_tc_tiling_on_sc=False)

    @functools.partial(
        pl.kernel,
        compiler_params=cp,
        out_type=jax.ShapeDtypeStruct((batch,), jnp.float32),
        mesh=mesh,
        scratch_types=[
            pltpu.VMEM((chunks, IDX_CHUNK), jnp.int32),
            pltpu.VMEM((chunks, IDX_CHUNK), jnp.int32),
            pltpu.VMEM((chunks, IDX_CHUNK), jnp.int32),
            pltpu.VMEM((per_w, d), jnp.float32),
            pltpu.VMEM((per_w, d), jnp.float32),
            pltpu.VMEM((per_w, d), jnp.float32),
            pltpu.VMEM((per_w * L,), jnp.float32),
            pltpu.VMEM((per_w,), jnp.float32),
            pltpu.SemaphoreType.DMA,
        ],
    )
    def k(head_hbm, rel_hbm, tail_hbm, ent_hbm, relemb_hbm, out_hbm,
          hi_v, ri_v, ti_v, h_v, r_v, t_v, p_v, o_v, sem):
        wid = lax.axis_index("s") * NC + lax.axis_index("c")
        base = wid * per_w

        for j in range(chunks):
            src = pl.ds(base + j * IDX_CHUNK, IDX_CHUNK)
            pltpu.sync_copy(head_hbm.at[src], hi_v.at[j])
            pltpu.sync_copy(rel_hbm.at[src], ri_v.at[j])
            pltpu.sync_copy(tail_hbm.at[src], ti_v.at[j])

        copies = []
        for j in range(chunks):
            dst = pl.ds(j * IDX_CHUNK, IDX_CHUNK)
            copies.append(pltpu.async_copy(ent_hbm.at[hi_v.at[j]], h_v.at[dst], sem))
            copies.append(pltpu.async_copy(relemb_hbm.at[ri_v.at[j]], r_v.at[dst], sem))
            copies.append(pltpu.async_copy(ent_hbm.at[ti_v.at[j]], t_v.at[dst], sem))
        for cp in copies:
            cp.wait()

        @pl.loop(0, per_w)
        def _(i):
            acc = h_v[i, pl.ds(0, L)] * r_v[i, pl.ds(0, L)] * t_v[i, pl.ds(0, L)]
            for c in range(1, d // L):
                sl = pl.ds(c * L, L)
                acc = acc + h_v[i, sl] * r_v[i, sl] * t_v[i, sl]
            p_v[pl.ds(i * L, L)] = acc

        iota = lax.iota(jnp.int32, L)

        @pl.loop(0, per_w // L)
        def _(g):
            bidx = g * (L * L) + iota * L
            acc = plsc.load_gather(p_v, [bidx])
            for kk in range(1, L):
                acc = acc + plsc.load_gather(p_v, [bidx + kk])
            o_v[pl.ds(g * L, L)] = acc

        pltpu.sync_copy(o_v, out_hbm.at[pl.ds(base, per_w)])

    return k


def kernel(head, relation, tail, entity_embeddings, relation_embeddings):
    batch = head.shape[0]
    d = entity_embeddings.shape[1]
    k = _make_sc_kernel(batch, d)
    return k(head.astype(jnp.int32), relation.astype(jnp.int32),
             tail.astype(jnp.int32), entity_embeddings, relation_embeddings)
